# baseline (speedup 1.0000x reference)
import jax
import jax.numpy as jnp
from jax import lax
from jax.experimental import pallas as pl
from jax.experimental.pallas import tpu as pltpu
from jax.experimental.pallas import tpu_sc as plsc

B = 16384
DIM = 16
NROWS = 1000000
NC = 2
NS = 16
NW = NC * NS
BPW = B // NW
CHUNK = 128
NCH = BPW // CHUNK
FLAT = DIM * NROWS
TPB = 8000512
C_D = [(d // 8) * TPB + (d % 8) * 128 for d in range(DIM)]


def _sc_body(uid_h, pid_h, nid_h, uembT_h, iembT_h, ubias_h, ibias_h, gb_h,
             pos_h, neg_h,
             uid_v, pid_v, nid_v,
             u_v, p_v, n_v, ub_v, pb_v, nb_v, gb_v, pos_v, neg_v, sem):
  wid = lax.axis_index("s") * NC + lax.axis_index("c")
  base = wid * BPW

  pltpu.sync_copy(gb_h, gb_v)

  for j in range(NCH):
    sl_h = pl.ds(base + j * CHUNK, CHUNK)
    pltpu.sync_copy(uid_h.at[sl_h], uid_v.at[j])
    pltpu.sync_copy(pid_h.at[sl_h], pid_v.at[j])
    pltpu.sync_copy(nid_h.at[sl_h], nid_v.at[j])


  copies = []
  for j in range(NCH):
    sl = pl.ds(j * CHUNK, CHUNK)
    copies.append(pltpu.make_async_copy(
        ubias_h.at[uid_v.at[j]], ub_v.at[sl], sem))
    copies.append(pltpu.make_async_copy(
        ibias_h.at[pid_v.at[j]], pb_v.at[sl], sem))
    copies.append(pltpu.make_async_copy(
        ibias_h.at[nid_v.at[j]], nb_v.at[sl], sem))
    for d in range(DIM):
      copies.append(pltpu.make_async_copy(
          uembT_h.at[d].at[uid_v.at[j]], u_v.at[d, sl], sem))
      copies.append(pltpu.make_async_copy(
          iembT_h.at[d].at[pid_v.at[j]], p_v.at[d, sl], sem))
      copies.append(pltpu.make_async_copy(
          iembT_h.at[d].at[nid_v.at[j]], n_v.at[d, sl], sem))
  for c in copies:
    c.start()
  for c in copies:
    c.wait()

  gbv = gb_v[...]

  def group(s, carry):
    sl = pl.ds(s * 16, 16)
    ubv = ub_v[sl]
    pos = gbv + ubv + pb_v[sl]
    neg = gbv + ubv + nb_v[sl]
    for d in range(DIM):
      ud = u_v[d, sl]
      pos = pos + ud * p_v[d, sl]
      neg = neg + ud * n_v[d, sl]
    pos_v[sl] = pos
    neg_v[sl] = neg
    return carry

  lax.fori_loop(0, BPW // 16, group, 0)

  pltpu.sync_copy(pos_v, pos_h.at[pl.ds(base, BPW)])
  pltpu.sync_copy(neg_v, neg_h.at[pl.ds(base, BPW)])


def kernel(user_ids, pos_item_ids, neg_item_ids, user_emb_w, item_emb_w,
           user_bias_w, item_bias_w, global_bias):
  gb16 = jnp.broadcast_to(global_bias.astype(jnp.float32), (16,))
  user_emb_p, item_emb_p = lax.optimization_barrier((user_emb_w, item_emb_w))
  uembT = user_emb_p.T
  iembT = item_emb_p.T
  ubias_flat = user_bias_w.reshape(-1)
  ibias_flat = item_bias_w.reshape(-1)
  mesh = plsc.VectorSubcoreMesh(core_axis_name="c", subcore_axis_name="s",
                                num_cores=NC, num_subcores=NS)
  f = pl.kernel(
      _sc_body,
      out_type=(jax.ShapeDtypeStruct((B,), jnp.float32),
                jax.ShapeDtypeStruct((B,), jnp.float32)),
      mesh=mesh,
      compiler_params=pltpu.CompilerParams(needs_layout_passes=False,
                                           use_tc_tiling_on_sc=False),
      scratch_types=[
          pltpu.VMEM((NCH, CHUNK), jnp.int32),
          pltpu.VMEM((NCH, CHUNK), jnp.int32),
          pltpu.VMEM((NCH, CHUNK), jnp.int32),
          pltpu.VMEM((DIM, BPW), jnp.float32),
          pltpu.VMEM((DIM, BPW), jnp.float32),
          pltpu.VMEM((DIM, BPW), jnp.float32),
          pltpu.VMEM((BPW,), jnp.float32),
          pltpu.VMEM((BPW,), jnp.float32),
          pltpu.VMEM((BPW,), jnp.float32),
          pltpu.VMEM((16,), jnp.float32),
          pltpu.VMEM((BPW,), jnp.float32),
          pltpu.VMEM((BPW,), jnp.float32),
          pltpu.SemaphoreType.DMA,
      ],
  )
  return f(user_ids, pos_item_ids, neg_item_ids, uembT, iembT,
           ubias_flat, ibias_flat, gb16)


# speedup vs baseline: 3.2894x; 3.2894x over previous
import jax
import jax.numpy as jnp
from jax import lax
from jax.experimental import pallas as pl
from jax.experimental.pallas import tpu as pltpu
from jax.experimental.pallas import tpu_sc as plsc

B = 16384
DIM = 16
NC = 2
NS = 16
NW = NC * NS
BPW = B // NW
CHUNK = 128
NCH = BPW // CHUNK
NG = BPW // 16


def _sc_body(uid_h, pid_h, nid_h, uemb_h, iemb_h, ubias_h, ibias_h, gb_h,
             pos_h, neg_h,
             uid_v, pid_v, nid_v, urows, prows, nrows,
             ub_v, pb_v, nb_v, gb_v, pos_v, neg_v, sem):
  wid = lax.axis_index("s") * NC + lax.axis_index("c")
  base = wid * BPW

  pltpu.sync_copy(gb_h, gb_v)

  for j in range(NCH):
    sl_h = pl.ds(base + j * CHUNK, CHUNK)
    pltpu.sync_copy(uid_h.at[sl_h], uid_v.at[j])
    pltpu.sync_copy(pid_h.at[sl_h], pid_v.at[j])
    pltpu.sync_copy(nid_h.at[sl_h], nid_v.at[j])

  copies = []
  for j in range(NCH):
    sl = pl.ds(j * CHUNK, CHUNK)
    copies.append(pltpu.make_async_copy(uemb_h.at[uid_v.at[j]], urows.at[sl], sem))
    copies.append(pltpu.make_async_copy(iemb_h.at[pid_v.at[j]], prows.at[sl], sem))
    copies.append(pltpu.make_async_copy(iemb_h.at[nid_v.at[j]], nrows.at[sl], sem))
    copies.append(pltpu.make_async_copy(ubias_h.at[uid_v.at[j]], ub_v.at[sl], sem))
    copies.append(pltpu.make_async_copy(ibias_h.at[pid_v.at[j]], pb_v.at[sl], sem))
    copies.append(pltpu.make_async_copy(ibias_h.at[nid_v.at[j]], nb_v.at[sl], sem))
  for c in copies:
    c.start()
  for c in copies:
    c.wait()

  gbv = gb_v[...]
  iota16 = lax.iota(jnp.int32, 16)
  lane_masks = [iota16 == i for i in range(16)]

  def group(g, carry):
    rbase = g * 16
    ubv = ub_v[pl.ds(rbase, 16)]
    pbv = pb_v[pl.ds(rbase, 16)]
    nbv = nb_v[pl.ds(rbase, 16)]
    pos = gbv + ubv + pbv
    neg = gbv + ubv + nbv
    for i in range(16):
      r = rbase + i
      u = urows[r, :]
      p = prows[r, :]
      n = nrows[r, :]
      dp = jnp.sum(u * p)
      dn = jnp.sum(u * n)
      pos = pos + jnp.where(lane_masks[i], dp, 0.0)
      neg = neg + jnp.where(lane_masks[i], dn, 0.0)
    pos_v[pl.ds(rbase, 16)] = pos
    neg_v[pl.ds(rbase, 16)] = neg
    return carry

  lax.fori_loop(0, NG, group, 0)

  pltpu.sync_copy(pos_v, pos_h.at[pl.ds(base, BPW)])
  pltpu.sync_copy(neg_v, neg_h.at[pl.ds(base, BPW)])


def kernel(user_ids, pos_item_ids, neg_item_ids, user_emb_w, item_emb_w,
           user_bias_w, item_bias_w, global_bias):
  gb16 = jnp.broadcast_to(global_bias.astype(jnp.float32), (16,))
  ubias_flat = user_bias_w.reshape(-1)
  ibias_flat = item_bias_w.reshape(-1)
  mesh = plsc.VectorSubcoreMesh(core_axis_name="c", subcore_axis_name="s",
                                num_cores=NC, num_subcores=NS)
  f = pl.kernel(
      _sc_body,
      out_type=(jax.ShapeDtypeStruct((B,), jnp.float32),
                jax.ShapeDtypeStruct((B,), jnp.float32)),
      mesh=mesh,
      compiler_params=pltpu.CompilerParams(needs_layout_passes=False,
                                           use_tc_tiling_on_sc=False),
      scratch_types=[
          pltpu.VMEM((NCH, CHUNK), jnp.int32),
          pltpu.VMEM((NCH, CHUNK), jnp.int32),
          pltpu.VMEM((NCH, CHUNK), jnp.int32),
          pltpu.VMEM((BPW, DIM), jnp.float32),
          pltpu.VMEM((BPW, DIM), jnp.float32),
          pltpu.VMEM((BPW, DIM), jnp.float32),
          pltpu.VMEM((BPW,), jnp.float32),
          pltpu.VMEM((BPW,), jnp.float32),
          pltpu.VMEM((BPW,), jnp.float32),
          pltpu.VMEM((16,), jnp.float32),
          pltpu.VMEM((BPW,), jnp.float32),
          pltpu.VMEM((BPW,), jnp.float32),
          pltpu.SemaphoreType.DMA,
      ],
  )
  return f(user_ids, pos_item_ids, neg_item_ids, user_emb_w, item_emb_w,
           ubias_flat, ibias_flat, gb16)


# speedup vs baseline: 6.4531x; 1.9618x over previous
import jax
import jax.numpy as jnp
from jax import lax
from jax.experimental import pallas as pl
from jax.experimental.pallas import tpu as pltpu
from jax.experimental.pallas import tpu_sc as plsc

B = 16384
DIM = 16
NROWS = 1000000
NC = 2
NS = 16
NW = NC * NS
BPW = B // NW
CHUNK = 128
NCH = BPW // CHUNK
W = 2048
BLK = DIM * W
NFULL = NROWS // W
NLAST = NROWS - 64
WT = NLAST - NFULL * W
NSLAB = NFULL + 1
SIZE = NSLAB * BLK
NT = 64
KMAX = (NSLAB + NW - 1) // NW


def _detile_body(uembT_h, iembT_h, udet_h, idet_h, slab_v, flat_v, sem):
  wid = lax.axis_index("s") * NC + lax.axis_index("c")

  def repack(width):
    nv = width // 16
    def body(i, carry):
      d = lax.div(i, nv)
      x = lax.rem(i, nv)
      xo = x * 16
      flat_v[pl.ds(d * W + xo, 16)] = slab_v[d, pl.ds(xo, 16)]
      return carry
    lax.fori_loop(0, DIM * nv, body, 0, unroll=8)

  for src_h, dst_h in ((uembT_h, udet_h), (iembT_h, idet_h)):
    for k in range(KMAX):
      job = wid + NW * k
      @pl.when(job < NFULL)
      def _(src_h=src_h, dst_h=dst_h, job=job):
        c = job * W
        pltpu.sync_copy(src_h.at[:, pl.ds(c, W)], slab_v)
        repack(W)
        pltpu.sync_copy(flat_v, dst_h.at[pl.ds(job * BLK, BLK)])
    @pl.when(wid == (8 if dst_h is udet_h else 9))
    def _(src_h=src_h, dst_h=dst_h):
      pltpu.sync_copy(src_h.at[:, pl.ds(NFULL * W, WT)],
                      slab_v.at[:, pl.ds(0, WT)])
      repack(WT)
      pltpu.sync_copy(flat_v, dst_h.at[pl.ds(NFULL * BLK, BLK)])


def _score_body(uid_h, pid_h, nid_h, udet_h, idet_h, ubias_h, ibias_h,
                utail_h, itail_h, gb_h,
                pos_h, neg_h,
                uid_v, pid_v, nid_v, ubase_v, pbase_v, nbase_v,
                u_v, p_v, n_v, ub_v, pb_v, nb_v, gb_v, ut_v, it_v,
                pos_v, neg_v, sem):
  wid = lax.axis_index("s") * NC + lax.axis_index("c")
  base = wid * BPW

  pltpu.sync_copy(gb_h, gb_v)
  pltpu.sync_copy(utail_h, ut_v)
  pltpu.sync_copy(itail_h, it_v)

  for j in range(NCH):
    sl_h = pl.ds(base + j * CHUNK, CHUNK)
    pltpu.sync_copy(uid_h.at[sl_h], uid_v.at[j])
    pltpu.sync_copy(pid_h.at[sl_h], pid_v.at[j])
    pltpu.sync_copy(nid_h.at[sl_h], nid_v.at[j])

  for j in range(NCH):
    def mkbase(s, carry):
      sl = pl.ds(s * 16, 16)
      for ids, bases in ((uid_v, ubase_v), (pid_v, pbase_v), (nid_v, nbase_v)):
        i = jnp.minimum(ids[j, sl], NLAST - 1)
        bases[j, sl] = i + lax.shift_right_logical(i, 11) * (BLK - W)
      return carry
    lax.fori_loop(0, CHUNK // 16, mkbase, 0)

  copies = []
  for j in range(NCH):
    sl = pl.ds(j * CHUNK, CHUNK)
    copies.append(pltpu.make_async_copy(
        ubias_h.at[uid_v.at[j]], ub_v.at[sl], sem))
    copies.append(pltpu.make_async_copy(
        ibias_h.at[pid_v.at[j]], pb_v.at[sl], sem))
    copies.append(pltpu.make_async_copy(
        ibias_h.at[nid_v.at[j]], nb_v.at[sl], sem))
    for d in range(DIM):
      usrc = udet_h.at[pl.ds(d * W, SIZE - d * W)]
      isrc = idet_h.at[pl.ds(d * W, SIZE - d * W)]
      copies.append(pltpu.make_async_copy(
          usrc.at[ubase_v.at[j]], u_v.at[d, sl], sem))
      copies.append(pltpu.make_async_copy(
          isrc.at[pbase_v.at[j]], p_v.at[d, sl], sem))
      copies.append(pltpu.make_async_copy(
          isrc.at[nbase_v.at[j]], n_v.at[d, sl], sem))
  for c in copies:
    c.start()
  for c in copies:
    c.wait()

  gbv = gb_v[...]
  zero = jnp.zeros((16,), jnp.int32)

  def group(s, carry):
    j2 = lax.shift_right_logical(s, 3)
    s2 = lax.bitwise_and(s, 7)
    sl16 = pl.ds(s2 * 16, 16)
    sl = pl.ds(s * 16, 16)
    uids = uid_v[j2, sl16]
    pids = pid_v[j2, sl16]
    nids = nid_v[j2, sl16]
    um = uids >= NLAST
    pm = pids >= NLAST
    nm = nids >= NLAST
    ut = jnp.maximum(uids - NLAST, zero)
    pt = jnp.maximum(pids - NLAST, zero)
    nt = jnp.maximum(nids - NLAST, zero)
    ubv = ub_v[sl]
    pos = gbv + ubv + pb_v[sl]
    neg = gbv + ubv + nb_v[sl]
    for d in range(DIM):
      ud = jnp.where(um, plsc.load_gather(ut_v, [ut + d * NT]), u_v[d, sl])
      pd = jnp.where(pm, plsc.load_gather(it_v, [pt + d * NT]), p_v[d, sl])
      nd = jnp.where(nm, plsc.load_gather(it_v, [nt + d * NT]), n_v[d, sl])
      pos = pos + ud * pd
      neg = neg + ud * nd
    pos_v[sl] = pos
    neg_v[sl] = neg
    return carry

  lax.fori_loop(0, BPW // 16, group, 0)

  pltpu.sync_copy(pos_v, pos_h.at[pl.ds(base, BPW)])
  pltpu.sync_copy(neg_v, neg_h.at[pl.ds(base, BPW)])


def kernel(user_ids, pos_item_ids, neg_item_ids, user_emb_w, item_emb_w,
           user_bias_w, item_bias_w, global_bias):
  gb16 = jnp.broadcast_to(global_bias.astype(jnp.float32), (16,))
  uembT = user_emb_w.T
  iembT = item_emb_w.T
  ubias_flat = user_bias_w.reshape(-1)
  ibias_flat = item_bias_w.reshape(-1)
  utail = user_emb_w[NLAST:, :].T.reshape(-1)
  itail = item_emb_w[NLAST:, :].T.reshape(-1)
  mesh = plsc.VectorSubcoreMesh(core_axis_name="c", subcore_axis_name="s",
                                num_cores=NC, num_subcores=NS)
  params = pltpu.CompilerParams(needs_layout_passes=False)

  detile = pl.kernel(
      _detile_body,
      out_type=(jax.ShapeDtypeStruct((SIZE,), jnp.float32),
                jax.ShapeDtypeStruct((SIZE,), jnp.float32)),
      mesh=mesh,
      compiler_params=params,
      scratch_types=[
          pltpu.VMEM((DIM, W), jnp.float32),
          pltpu.VMEM((BLK,), jnp.float32),
          pltpu.SemaphoreType.DMA,
      ],
  )
  udet, idet = detile(uembT, iembT)

  score = pl.kernel(
      _score_body,
      out_type=(jax.ShapeDtypeStruct((B,), jnp.float32),
                jax.ShapeDtypeStruct((B,), jnp.float32)),
      mesh=mesh,
      compiler_params=params,
      scratch_types=[
          pltpu.VMEM((NCH, CHUNK), jnp.int32),
          pltpu.VMEM((NCH, CHUNK), jnp.int32),
          pltpu.VMEM((NCH, CHUNK), jnp.int32),
          pltpu.VMEM((NCH, CHUNK), jnp.int32),
          pltpu.VMEM((NCH, CHUNK), jnp.int32),
          pltpu.VMEM((NCH, CHUNK), jnp.int32),
          pltpu.VMEM((DIM, BPW), jnp.float32),
          pltpu.VMEM((DIM, BPW), jnp.float32),
          pltpu.VMEM((DIM, BPW), jnp.float32),
          pltpu.VMEM((BPW,), jnp.float32),
          pltpu.VMEM((BPW,), jnp.float32),
          pltpu.VMEM((BPW,), jnp.float32),
          pltpu.VMEM((16,), jnp.float32),
          pltpu.VMEM((DIM * NT,), jnp.float32),
          pltpu.VMEM((DIM * NT,), jnp.float32),
          pltpu.VMEM((BPW,), jnp.float32),
          pltpu.VMEM((BPW,), jnp.float32),
          pltpu.SemaphoreType.DMA,
      ],
  )
  return score(user_ids, pos_item_ids, neg_item_ids, udet, idet,
               ubias_flat, ibias_flat, utail, itail, gb16)
